# Initial kernel scaffold; baseline (speedup 1.0000x reference)
import jax
import jax.numpy as jnp
from jax.experimental import pallas as pl


def kernel(x):
    raise NotImplementedError("write your pallas kernel here")


# speedup vs baseline: 9.1475x; 9.1475x over previous
import math

import jax
import jax.numpy as jnp
from jax.experimental import pallas as pl

_SR = 0.1


def _sparsify_block(x_ref, o_ref, *, k, n):
    x = x_ref[...]
    bits = jax.lax.bitcast_convert_type(x, jnp.int32)
    mask31 = jnp.int32(0x7FFFFFFF)
    s = jnp.where(bits >= 0, bits, bits ^ mask31)

    def body(i, t):
        bit = jnp.left_shift(jnp.int32(1), jnp.int32(31) - i)
        cand = t + bit
        cnt = jnp.sum((s >= cand).astype(jnp.int32), axis=-1, keepdims=True)
        return jnp.where(cnt >= k, cand, t)

    t0 = jnp.full((x.shape[0], 1), jnp.int32(-2147483648))
    t = jax.lax.fori_loop(0, 32, body, t0)
    tb = jnp.where(t >= 0, t, t ^ mask31)
    thr = jax.lax.bitcast_convert_type(tb, jnp.float32)
    res = jnp.where(x >= thr, x, jnp.float32(0.0))
    denom = jnp.sum(res, axis=-1, keepdims=True) / jnp.float32(n)
    o_ref[...] = res / denom


def kernel(x):
    b, n = x.shape
    k = int(math.ceil(_SR * n))
    rows = 8 if b % 8 == 0 else b
    grid = (b // rows,)
    import functools

    body = functools.partial(_sparsify_block, k=k, n=n)
    return pl.pallas_call(
        body,
        grid=grid,
        in_specs=[pl.BlockSpec((rows, n), lambda i: (i, 0))],
        out_specs=pl.BlockSpec((rows, n), lambda i: (i, 0)),
        out_shape=jax.ShapeDtypeStruct((b, n), jnp.float32),
    )(x)
